# baseline (device time: 52024 ns/iter reference)
import jax
import jax.numpy as jnp
from jax import lax
from jax.experimental import pallas as pl
from jax.experimental.pallas import tpu as pltpu

N_DEV = 8
B_LOC = 2
SQ = 128
D = 512
H_LOC = 8
DH = 64
SCALE = 0.125


def kernel(x, Wq, Wo, Wk, Wv):
    def body(x_ref, wq_ref, wo_ref, wk_ref, wv_ref, out_ref,
             x_all, d_send, d_recv, acc, wqkv_bf, wo_bf,
             ag_send, ag_recv, rs_send, rs_recv):
        my = lax.axis_index("i")

        barrier_sem = pltpu.get_barrier_semaphore()
        for k in range(1, N_DEV):
            pl.semaphore_signal(
                barrier_sem, inc=1,
                device_id=(lax.rem(my + k, N_DEV),),
                device_id_type=pl.DeviceIdType.MESH,
            )
        pl.semaphore_wait(barrier_sem, N_DEV - 1)

        for b in range(B_LOC):
            x_all[my, b] = x_ref[b].T.astype(jnp.bfloat16)
        pending = []
        for k in range(1, N_DEV):
            rdma = pltpu.make_async_remote_copy(
                src_ref=x_all.at[my], dst_ref=x_all.at[my],
                send_sem=ag_send.at[k - 1], recv_sem=ag_recv.at[my],
                device_id=(lax.rem(my + k, N_DEV),),
                device_id_type=pl.DeviceIdType.MESH,
            )
            rdma.start()
            pending.append(rdma)

        wqkv_bf[:, 0:512] = wq_ref[...].astype(jnp.bfloat16)
        wqkv_bf[:, 512:1024] = wk_ref[...].astype(jnp.bfloat16)
        wqkv_bf[:, 1024:1536] = wv_ref[...].astype(jnp.bfloat16)
        wo_bf[...] = wo_ref[...].astype(jnp.bfloat16)

        def compute_chunk(slot, b):
            xt = x_all[slot, b]
            qkvt = lax.dot_general(
                wqkv_bf[...], xt, (((0,), (0,)), ((), ())),
                preferred_element_type=jnp.float32,
            )
            qt = qkvt[0:512].reshape(H_LOC, DH, SQ).astype(jnp.bfloat16)
            kt = qkvt[512:1024].reshape(H_LOC, DH, SQ).astype(jnp.bfloat16)
            vt = qkvt[1024:1536].reshape(H_LOC, DH, SQ).astype(jnp.bfloat16)
            s = lax.dot_general(
                qt, kt, (((1,), (1,)), ((0,), (0,))),
                preferred_element_type=jnp.float32,
            ) * SCALE
            m = jnp.max(s, axis=2, keepdims=True)
            p = jnp.exp(s - m)
            l = jnp.sum(p, axis=2)
            ot = lax.dot_general(
                vt, p.astype(jnp.bfloat16), (((2,), (2,)), ((0,), (0,))),
                preferred_element_type=jnp.float32,
            )
            ot = ot / l[:, None, :]
            return lax.dot_general(
                wo_bf[...], ot.astype(jnp.bfloat16).reshape(H_LOC * DH, SQ),
                (((0,), (0,)), ((), ())),
                preferred_element_type=jnp.float32,
            )

        for b in range(B_LOC):
            acc[b] = compute_chunk(my, b)

        def fold_delta(j):
            src = lax.rem(my + j, N_DEV)
            pltpu.make_async_remote_copy(
                src_ref=d_recv.at[src], dst_ref=d_recv.at[src],
                send_sem=rs_send.at[j - 1], recv_sem=rs_recv.at[src],
                device_id=(src,), device_id_type=pl.DeviceIdType.MESH,
            ).wait_recv()
            for b in range(B_LOC):
                acc[b] = acc[b] + d_recv[src, b].astype(jnp.float32)

        for k in range(1, N_DEV):
            src = lax.rem(my + (N_DEV - k), N_DEV)
            pltpu.make_async_remote_copy(
                src_ref=x_all.at[src], dst_ref=x_all.at[src],
                send_sem=ag_send.at[k - 1], recv_sem=ag_recv.at[src],
                device_id=(src,), device_id_type=pl.DeviceIdType.MESH,
            ).wait_recv()
            for b in range(B_LOC):
                d_send[k - 1, b] = compute_chunk(src, b).astype(jnp.bfloat16)
            rdma = pltpu.make_async_remote_copy(
                src_ref=d_send.at[k - 1], dst_ref=d_recv.at[my],
                send_sem=rs_send.at[k - 1], recv_sem=rs_recv.at[my],
                device_id=(src,),
                device_id_type=pl.DeviceIdType.MESH,
            )
            rdma.start()
            pending.append(rdma)
            if k >= 3:
                fold_delta(k - 2)

        fold_delta(N_DEV - 2)
        fold_delta(N_DEV - 1)

        for b in range(B_LOC):
            out_ref[b] = acc[b].T

        for rdma in pending:
            rdma.wait_send()

    return pl.pallas_call(
        body,
        out_shape=jax.ShapeDtypeStruct((B_LOC, SQ, D), jnp.float32),
        in_specs=[pl.BlockSpec(memory_space=pltpu.VMEM)] * 5,
        out_specs=pl.BlockSpec(memory_space=pltpu.VMEM),
        scratch_shapes=[
            pltpu.VMEM((N_DEV, B_LOC, D, SQ), jnp.bfloat16),
            pltpu.VMEM((N_DEV - 1, B_LOC, D, SQ), jnp.bfloat16),
            pltpu.VMEM((N_DEV, B_LOC, D, SQ), jnp.bfloat16),
            pltpu.VMEM((B_LOC, D, SQ), jnp.float32),
            pltpu.VMEM((D, 3 * H_LOC * DH), jnp.bfloat16),
            pltpu.VMEM((H_LOC * DH, D), jnp.bfloat16),
            pltpu.SemaphoreType.DMA((N_DEV - 1,)),
            pltpu.SemaphoreType.DMA((N_DEV,)),
            pltpu.SemaphoreType.DMA((N_DEV - 1,)),
            pltpu.SemaphoreType.DMA((N_DEV,)),
        ],
        compiler_params=pltpu.CompilerParams(collective_id=0),
    )(x, Wq, Wo, Wk, Wv)


# device time: 46312 ns/iter; 1.1233x vs baseline; 1.1233x over previous
import jax
import jax.numpy as jnp
from jax import lax
from jax.experimental import pallas as pl
from jax.experimental.pallas import tpu as pltpu

N_DEV = 8
B_LOC = 2
SQ = 128
D = 512
H_LOC = 8
DH = 64
SCALE = 0.125


def kernel(x, Wq, Wo, Wk, Wv):
    def body(x_ref, wq_ref, wo_ref, wk_ref, wv_ref, out_ref,
             x_all, d_send, d_recv, acc, wqkv_bf, wo_bf,
             ag_send, ag_recv, rs_send, rs_recv):
        my = lax.axis_index("i")

        barrier_sem = pltpu.get_barrier_semaphore()
        for k in range(1, N_DEV):
            pl.semaphore_signal(
                barrier_sem, inc=1,
                device_id=(lax.rem(my + k, N_DEV),),
                device_id_type=pl.DeviceIdType.MESH,
            )
        pl.semaphore_wait(barrier_sem, N_DEV - 1)

        for b in range(B_LOC):
            x_all[my, b] = x_ref[b].T.astype(jnp.bfloat16)
        pending = []
        for k in range(1, N_DEV):
            rdma = pltpu.make_async_remote_copy(
                src_ref=x_all.at[my], dst_ref=x_all.at[my],
                send_sem=ag_send.at[k - 1], recv_sem=ag_recv.at[my],
                device_id=(lax.rem(my + k, N_DEV),),
                device_id_type=pl.DeviceIdType.MESH,
            )
            rdma.start()
            pending.append(rdma)

        wqkv_bf[:, 0:512] = wq_ref[...].astype(jnp.bfloat16)
        wqkv_bf[:, 512:1024] = wk_ref[...].astype(jnp.bfloat16)
        wqkv_bf[:, 1024:1536] = wv_ref[...].astype(jnp.bfloat16)
        wo_bf[...] = wo_ref[...].astype(jnp.bfloat16)

        def compute_chunk(slot, b):
            xt = x_all[slot, b]
            qkvt = lax.dot_general(
                wqkv_bf[...], xt, (((0,), (0,)), ((), ())),
                preferred_element_type=jnp.float32,
            )
            qt = qkvt[0:512].reshape(H_LOC, DH, SQ).astype(jnp.bfloat16)
            kt = qkvt[512:1024].reshape(H_LOC, DH, SQ).astype(jnp.bfloat16)
            vt = qkvt[1024:1536].reshape(H_LOC, DH, SQ).astype(jnp.bfloat16)
            s = lax.dot_general(
                qt, kt, (((1,), (1,)), ((0,), (0,))),
                preferred_element_type=jnp.float32,
            ) * SCALE
            m = jnp.max(s, axis=2, keepdims=True)
            p = jnp.exp(s - m)
            l = jnp.sum(p, axis=2)
            ot = lax.dot_general(
                vt, p.astype(jnp.bfloat16), (((2,), (2,)), ((0,), (0,))),
                preferred_element_type=jnp.float32,
            )
            ot = ot / l[:, None, :]
            return lax.dot_general(
                wo_bf[...], ot.astype(jnp.bfloat16).reshape(H_LOC * DH, SQ),
                (((0,), (0,)), ((), ())),
                preferred_element_type=jnp.float32,
            )

        for b in range(B_LOC):
            acc[b] = compute_chunk(my, b)

        def fold_delta(j):
            src = lax.rem(my + j, N_DEV)
            pltpu.make_async_remote_copy(
                src_ref=d_recv.at[src], dst_ref=d_recv.at[src],
                send_sem=rs_send.at[j - 1], recv_sem=rs_recv.at[src],
                device_id=(src,), device_id_type=pl.DeviceIdType.MESH,
            ).wait_recv()
            for b in range(B_LOC):
                acc[b] = acc[b] + d_recv[src, b].astype(jnp.float32)

        for k in range(1, N_DEV):
            src = lax.rem(my + (N_DEV - k), N_DEV)
            pltpu.make_async_remote_copy(
                src_ref=x_all.at[src], dst_ref=x_all.at[src],
                send_sem=ag_send.at[k - 1], recv_sem=ag_recv.at[src],
                device_id=(src,), device_id_type=pl.DeviceIdType.MESH,
            ).wait_recv()
            for b in range(B_LOC):
                d_send[k - 1, b] = compute_chunk(src, b).astype(jnp.bfloat16)
            rdma = pltpu.make_async_remote_copy(
                src_ref=d_send.at[k - 1], dst_ref=d_recv.at[my],
                send_sem=rs_send.at[k - 1], recv_sem=rs_recv.at[my],
                device_id=(src,),
                device_id_type=pl.DeviceIdType.MESH,
            )
            rdma.start()
            pending.append(rdma)

        for j in range(1, N_DEV):
            fold_delta(j)

        for b in range(B_LOC):
            out_ref[b] = acc[b].T

        for rdma in pending:
            rdma.wait_send()

    return pl.pallas_call(
        body,
        out_shape=jax.ShapeDtypeStruct((B_LOC, SQ, D), jnp.float32),
        in_specs=[pl.BlockSpec(memory_space=pltpu.VMEM)] * 5,
        out_specs=pl.BlockSpec(memory_space=pltpu.VMEM),
        scratch_shapes=[
            pltpu.VMEM((N_DEV, B_LOC, D, SQ), jnp.bfloat16),
            pltpu.VMEM((N_DEV - 1, B_LOC, D, SQ), jnp.bfloat16),
            pltpu.VMEM((N_DEV, B_LOC, D, SQ), jnp.bfloat16),
            pltpu.VMEM((B_LOC, D, SQ), jnp.float32),
            pltpu.VMEM((D, 3 * H_LOC * DH), jnp.bfloat16),
            pltpu.VMEM((H_LOC * DH, D), jnp.bfloat16),
            pltpu.SemaphoreType.DMA((N_DEV - 1,)),
            pltpu.SemaphoreType.DMA((N_DEV,)),
            pltpu.SemaphoreType.DMA((N_DEV - 1,)),
            pltpu.SemaphoreType.DMA((N_DEV,)),
        ],
        compiler_params=pltpu.CompilerParams(collective_id=0),
    )(x, Wq, Wo, Wk, Wv)
